# baseline (speedup 1.0000x reference)
import functools

import jax
import jax.numpy as jnp
from jax import lax
from jax.experimental import pallas as pl
from jax.experimental.pallas import tpu as pltpu
from jax.experimental.pallas import tpu_sc as plsc

D_MODEL = 768
NUM_QUERIES = 512
BATCH = 4
SEQ = 8192
N_ROWS = BATCH * SEQ
ROW_CHUNK = 2048
N_SEL = BATCH * NUM_QUERIES

NC = 2
NS = 16
NW = NC * NS
B_PER_W = N_SEL // NW


def _score_body(x_ref, w_ref, b_ref, o_ref):
    x = x_ref[...]
    w = w_ref[...]
    s = jax.lax.dot_general(x, w, (((1,), (0,)), ((), ())),
                            preferred_element_type=jnp.float32)
    o_ref[...] = s + b_ref[0, 0]


def _scores(tokens_flat, w_col, b2):
    return pl.pallas_call(
        _score_body,
        grid=(N_ROWS // ROW_CHUNK,),
        in_specs=[
            pl.BlockSpec((ROW_CHUNK, D_MODEL), lambda i: (i, 0)),
            pl.BlockSpec((D_MODEL, 1), lambda i: (0, 0)),
            pl.BlockSpec((1, 1), lambda i: (0, 0)),
        ],
        out_specs=pl.BlockSpec((ROW_CHUNK, 1), lambda i: (i, 0)),
        out_shape=jax.ShapeDtypeStruct((N_ROWS, 1), jnp.float32),
    )(tokens_flat, w_col, b2)


def _gather_body(tokens_hbm, idx_hbm, out_hbm, idx_v, rows_v, sem):
    wid = lax.axis_index("c") * NS + lax.axis_index("s")
    base = wid * B_PER_W
    batch = wid // (NW // BATCH)
    pltpu.sync_copy(idx_hbm.at[pl.ds(base, B_PER_W)], idx_v)
    off = batch * SEQ
    for j in range(B_PER_W // 16):
        sl = pl.ds(j * 16, 16)
        idx_v[sl] = idx_v[sl] + off
    pltpu.async_copy(tokens_hbm.at[idx_v], rows_v, sem).wait()
    pltpu.sync_copy(rows_v, out_hbm.at[pl.ds(base, B_PER_W)])


_gather = functools.partial(
    pl.kernel,
    _gather_body,
    out_type=jax.ShapeDtypeStruct((N_SEL, D_MODEL), jnp.float32),
    mesh=plsc.VectorSubcoreMesh(core_axis_name="c", subcore_axis_name="s"),
    scratch_types=[
        pltpu.VMEM((B_PER_W,), jnp.int32),
        pltpu.VMEM((B_PER_W, D_MODEL), jnp.float32),
        pltpu.SemaphoreType.DMA,
    ],
)()


def kernel(tokens, W, b):
    tokens_flat = tokens.reshape(N_ROWS, D_MODEL)
    scores = _scores(tokens_flat, W.reshape(1, D_MODEL).T,
                     b.reshape(1, 1)).reshape(BATCH, SEQ)
    topk = jax.lax.top_k(scores, NUM_QUERIES)[1]
    sel = _gather(tokens_flat, topk.reshape(N_SEL))
    return sel.reshape(BATCH, NUM_QUERIES, D_MODEL), topk
